# baseline (speedup 1.0000x reference)
import jax
import jax.numpy as jnp
from jax import lax
from jax.experimental import pallas as pl
from jax.experimental.pallas import tpu as pltpu
from jax.experimental.pallas import tpu_sc as plsc

_MAX_SHIFT = 7
_NUM_XS = 2 * _MAX_SHIFT + 1
_N = 320000
_D = 128
_NEMB = 450
_NW = 32
_BPW = _N // _NW
_CH = 80
_NCH = _BPW // _CH
_NBUF = 5


def _idx_body(x_ref, idx_ref):
    x = x_ref[...].astype(jnp.float32)
    j = lax.broadcasted_iota(jnp.int32, (3 * _D, _D), 0)
    k = lax.broadcasted_iota(jnp.int32, (3 * _D, _D), 1)
    jm = j % 3
    w = jnp.where(jm == 0, 1.0, jnp.where(jm == 1, 30.0, 2.0))
    W = jnp.where(j // 3 == k, w, 0.0)
    idx_f = jax.lax.dot(x, W, preferred_element_type=jnp.float32)
    idx_ref[...] = (idx_f + 2.0 * (_MAX_SHIFT * _NUM_XS + _MAX_SHIFT)).astype(
        jnp.int32)


def _sc_body(idx_hbm, emb_ref, out_ref, idx_v, table_v, rows, gsem, ssem):
    wid = lax.axis_index("s") * 2 + lax.axis_index("c")
    ebase = wid * _BPW

    @pl.when(lax.axis_index("s") == 0)
    def _():
        pltpu.sync_copy(emb_ref, table_v)

    plsc.subcore_barrier()

    pltpu.sync_copy(idx_hbm.at[pl.ds(ebase, _BPW)], idx_v)

    def start_gather(b, c):
        idx_sl = idx_v.at[pl.ds(c * _CH, _CH)]
        pltpu.async_copy(table_v.at[idx_sl], rows.at[b], gsem.at[b])

    def wait_gather(b):
        pltpu.make_async_copy(
            out_ref.at[pl.ds(0, _CH)], rows.at[b], gsem.at[b]).wait()

    def start_store(b, c):
        pltpu.async_copy(
            rows.at[b], out_ref.at[pl.ds(ebase + c * _CH, _CH)], ssem.at[b])

    def wait_store(b):
        pltpu.make_async_copy(
            rows.at[b], out_ref.at[pl.ds(0, _CH)], ssem.at[b]).wait()

    def ch_body(p, carry):
        for b in range(_NBUF):
            c = p * _NBUF + b
            wait_gather(b)
            start_store(b, c)

            @pl.when(c + _NBUF < _NCH)
            def _():
                wait_store(b)
                start_gather(b, c + _NBUF)

        return carry

    for b in range(_NBUF):
        start_gather(b, b)
    lax.fori_loop(0, _NCH // _NBUF, ch_body, 0)
    for b in range(_NBUF):
        wait_store(b)


def kernel(x, emb):
    idx = pl.pallas_call(
        _idx_body,
        out_shape=jax.ShapeDtypeStruct((_N // _D, _D), jnp.int32),
    )(x.reshape(_N // _D, 3 * _D)).reshape(_N)

    mesh = plsc.VectorSubcoreMesh(core_axis_name="c", subcore_axis_name="s")
    gather = pl.kernel(
        _sc_body,
        out_type=jax.ShapeDtypeStruct((_N, _D), jnp.float32),
        mesh=mesh,
        compiler_params=pltpu.CompilerParams(needs_layout_passes=False),
        scratch_types=[
            pltpu.VMEM((_BPW,), jnp.int32),
            pltpu.VMEM_SHARED((_NEMB, _D), jnp.float32),
            pltpu.VMEM((_NBUF, _CH, _D), jnp.float32),
            pltpu.SemaphoreType.DMA((_NBUF,)),
            pltpu.SemaphoreType.DMA((_NBUF,)),
        ],
    )
    return gather(idx, emb)


# speedup vs baseline: 3.0839x; 3.0839x over previous
import jax
import jax.numpy as jnp
from jax import lax
from jax.experimental import pallas as pl
from jax.experimental.pallas import tpu as pltpu
from jax.experimental.pallas import tpu_sc as plsc

_MAX_SHIFT = 7
_NUM_XS = 2 * _MAX_SHIFT + 1
_N = 320000
_D = 128
_NEMB = 450
_NW = 32
_BPW = _N // _NW
_CH = 80
_NCH = _BPW // _CH
_NBUF = 5


def _idx_body(x_ref, idx_ref):
    x = x_ref[...]
    idx_ref[...] = (
        2 * ((x[1, :] + _MAX_SHIFT) * _NUM_XS + (x[2, :] + _MAX_SHIFT))
        + x[0, :])


def _sc_body(idx_hbm, emb_ref, out_ref, idx_v, table_v, rows, gsem, ssem):
    wid = lax.axis_index("s") * 2 + lax.axis_index("c")
    ebase = wid * _BPW

    pltpu.async_copy(idx_hbm.at[pl.ds(ebase, _BPW)], idx_v, gsem.at[0])

    @pl.when(lax.axis_index("s") == 0)
    def _():
        pltpu.sync_copy(emb_ref, table_v)

    plsc.subcore_barrier()
    pltpu.make_async_copy(
        idx_hbm.at[pl.ds(ebase, _BPW)], idx_v, gsem.at[0]).wait()

    def start_gather(b, c):
        idx_sl = idx_v.at[pl.ds(c * _CH, _CH)]
        pltpu.async_copy(table_v.at[idx_sl], rows.at[b], gsem.at[b])

    def wait_gather(b):
        pltpu.make_async_copy(
            out_ref.at[pl.ds(0, _CH)], rows.at[b], gsem.at[b]).wait()

    def start_store(b, c):
        pltpu.async_copy(
            rows.at[b], out_ref.at[pl.ds(ebase + c * _CH, _CH)], ssem.at[b])

    def wait_store(b):
        pltpu.make_async_copy(
            rows.at[b], out_ref.at[pl.ds(0, _CH)], ssem.at[b]).wait()

    def ch_body(p, carry):
        for b in range(_NBUF):
            c = p * _NBUF + b
            wait_gather(b)
            start_store(b, c)

            @pl.when(c + _NBUF < _NCH)
            def _():
                wait_store(b)
                start_gather(b, c + _NBUF)

        return carry

    for b in range(_NBUF):
        start_gather(b, b)
    lax.fori_loop(0, _NCH // _NBUF, ch_body, 0)
    for b in range(_NBUF):
        wait_store(b)


def kernel(x, emb):
    idx = pl.pallas_call(
        _idx_body,
        out_shape=jax.ShapeDtypeStruct((_N,), jnp.int32),
    )(x.T)

    mesh = plsc.VectorSubcoreMesh(core_axis_name="c", subcore_axis_name="s")
    gather = pl.kernel(
        _sc_body,
        out_type=jax.ShapeDtypeStruct((_N, _D), jnp.float32),
        mesh=mesh,
        compiler_params=pltpu.CompilerParams(needs_layout_passes=False),
        scratch_types=[
            pltpu.VMEM((_BPW,), jnp.int32),
            pltpu.VMEM_SHARED((_NEMB, _D), jnp.float32),
            pltpu.VMEM((_NBUF, _CH, _D), jnp.float32),
            pltpu.SemaphoreType.DMA((_NBUF,)),
            pltpu.SemaphoreType.DMA((_NBUF,)),
        ],
    )
    return gather(idx, emb)
